# baseline (device time: 22090 ns/iter reference)
import jax
import jax.numpy as jnp
from jax import lax
from jax.experimental import pallas as pl
from jax.experimental.pallas import tpu as pltpu

N_DEV = 4
N_LAYER = 3
B = 128
D = 128
H = 256


def kernel(x, Win0, Wout0, Win1, Wout1, Win2, Wout2):
    def body(x_ref, win0_ref, wout0_ref, win1_ref, wout1_ref, win2_ref,
             wout2_ref, out_ref, wfi, wfo, send_sems, recv_sems):
        my = lax.axis_index("i")
        wins = (win0_ref, win1_ref, win2_ref)
        wouts = (wout0_ref, wout1_ref, wout2_ref)

        barrier = pltpu.get_barrier_semaphore()
        for o in range(1, N_DEV):
            pl.semaphore_signal(
                barrier, inc=1,
                device_id=((my + o) % N_DEV,),
                device_id_type=pl.DeviceIdType.MESH,
            )
        for l in range(N_LAYER):
            wfi[pl.ds(l * N_DEV + my, 1)] = wins[l][...].astype(jnp.bfloat16)[None]
            wfo[pl.ds(l * N_DEV + my, 1)] = wouts[l][...].astype(jnp.bfloat16)[None]
        pl.semaphore_wait(barrier, N_DEV - 1)

        sends = []
        for o in range(1, N_DEV):
            tgt = (my + o) % N_DEV
            for l in range(N_LAYER):
                for kind, buf in ((0, wfi), (1, wfo)):
                    slot = buf.at[pl.ds(l * N_DEV + my, 1)]
                    rdma = pltpu.make_async_remote_copy(
                        src_ref=slot,
                        dst_ref=slot,
                        send_sem=send_sems.at[l * 2 + kind, o - 1],
                        recv_sem=recv_sems.at[l * 2 + kind, o - 1],
                        device_id=(tgt,),
                        device_id_type=pl.DeviceIdType.MESH,
                    )
                    rdma.start()
                    sends.append(rdma)

        for o in range(1, N_DEV):
            for l in range(N_LAYER):
                for kind, buf in ((0, wfi), (1, wfo)):
                    src = (my - o) % N_DEV
                    slot = buf.at[pl.ds(l * N_DEV + src, 1)]
                    recv = pltpu.make_async_remote_copy(
                        src_ref=slot,
                        dst_ref=slot,
                        send_sem=send_sems.at[l * 2 + kind, o - 1],
                        recv_sem=recv_sems.at[l * 2 + kind, o - 1],
                        device_id=(my,),
                        device_id_type=pl.DeviceIdType.MESH,
                    )
                    recv.wait_recv()

        xc = x_ref[...].astype(jnp.bfloat16)
        for l in range(N_LAYER):
            acc = jnp.zeros((B, D), jnp.float32)
            for d in range(N_DEV):
                h = jnp.dot(xc, wfi[l * N_DEV + d],
                            preferred_element_type=jnp.float32)
                h = jnp.maximum(h, 0.0)
                acc = acc + jnp.dot(h.astype(jnp.bfloat16),
                                    wfo[l * N_DEV + d],
                                    preferred_element_type=jnp.float32)
            xc = acc.astype(jnp.bfloat16)
        out_ref[...] = acc

        for s in sends:
            s.wait_send()

    return pl.pallas_call(
        body,
        out_shape=jax.ShapeDtypeStruct((B, D), jnp.float32),
        in_specs=[pl.BlockSpec(memory_space=pltpu.VMEM)] * 7,
        out_specs=pl.BlockSpec(memory_space=pltpu.VMEM),
        scratch_shapes=[
            pltpu.VMEM((N_LAYER * N_DEV, D, H), jnp.bfloat16),
            pltpu.VMEM((N_LAYER * N_DEV, H, D), jnp.bfloat16),
            pltpu.SemaphoreType.DMA((N_LAYER * 2, N_DEV - 1)),
            pltpu.SemaphoreType.DMA((N_LAYER * 2, N_DEV - 1)),
        ],
        compiler_params=pltpu.CompilerParams(collective_id=0),
    )(x, Win0, Wout0, Win1, Wout1, Win2, Wout2)


# device time: 21139 ns/iter; 1.0450x vs baseline; 1.0450x over previous
import os

import jax
import jax.numpy as jnp
from jax import lax
from jax.experimental import pallas as pl
from jax.experimental.pallas import tpu as pltpu

N_DEV = 4
N_LAYER = 3
B = 128
D = 128
H = 256

_WDT = os.environ.get("WDTYPE", "bf16")
_wire_dt = jnp.float8_e4m3fn if _WDT == "f8" else jnp.bfloat16


def kernel(x, Win0, Wout0, Win1, Wout1, Win2, Wout2):
    def body(x_ref, win0_ref, wout0_ref, win1_ref, wout1_ref, win2_ref,
             wout2_ref, out_ref, wfi, wfo, send_sems, recv_sems):
        my = lax.axis_index("i")
        wins = (win0_ref, win1_ref, win2_ref)
        wouts = (wout0_ref, wout1_ref, wout2_ref)

        barrier = pltpu.get_barrier_semaphore()
        for o in range(1, N_DEV):
            pl.semaphore_signal(
                barrier, inc=1,
                device_id=((my + o) % N_DEV,),
                device_id_type=pl.DeviceIdType.MESH,
            )

        for l in range(N_LAYER):
            wfi[pl.ds(l * N_DEV + my, 1)] = wins[l][...].astype(_wire_dt)[None]
            wfo[pl.ds(l * N_DEV + my, 1)] = wouts[l][...].astype(_wire_dt)[None]
        pl.semaphore_wait(barrier, N_DEV - 1)

        sends = []
        for l in range(N_LAYER):
            for o in range(1, N_DEV):
                tgt = (my + o) % N_DEV
                for kind, buf in ((0, wfi), (1, wfo)):
                    slot = buf.at[pl.ds(l * N_DEV + my, 1)]
                    rdma = pltpu.make_async_remote_copy(
                        src_ref=slot,
                        dst_ref=slot,
                        send_sem=send_sems.at[l * 2 + kind, o - 1],
                        recv_sem=recv_sems.at[l * 2 + kind, o - 1],
                        device_id=(tgt,),
                        device_id_type=pl.DeviceIdType.MESH,
                    )
                    rdma.start()
                    sends.append(rdma)

        xc = x_ref[...].astype(jnp.bfloat16)
        acc = None
        for l in range(N_LAYER):
            for o in range(1, N_DEV):
                for kind, buf in ((0, wfi), (1, wfo)):
                    src = (my - o) % N_DEV
                    slot = buf.at[pl.ds(l * N_DEV + src, 1)]
                    recv = pltpu.make_async_remote_copy(
                        src_ref=slot,
                        dst_ref=slot,
                        send_sem=send_sems.at[l * 2 + kind, o - 1],
                        recv_sem=recv_sems.at[l * 2 + kind, o - 1],
                        device_id=(my,),
                        device_id_type=pl.DeviceIdType.MESH,
                    )
                    recv.wait_recv()
            acc = jnp.zeros((B, D), jnp.float32)
            for d in range(N_DEV):
                h = jnp.dot(xc, wfi[l * N_DEV + d].astype(jnp.bfloat16),
                            preferred_element_type=jnp.float32)
                h = jnp.maximum(h, 0.0)
                acc = acc + jnp.dot(h.astype(jnp.bfloat16),
                                    wfo[l * N_DEV + d].astype(jnp.bfloat16),
                                    preferred_element_type=jnp.float32)
            xc = acc.astype(jnp.bfloat16)
        out_ref[...] = acc

        for s in sends:
            s.wait_send()

    return pl.pallas_call(
        body,
        out_shape=jax.ShapeDtypeStruct((B, D), jnp.float32),
        in_specs=[pl.BlockSpec(memory_space=pltpu.VMEM)] * 7,
        out_specs=pl.BlockSpec(memory_space=pltpu.VMEM),
        scratch_shapes=[
            pltpu.VMEM((N_LAYER * N_DEV, D, H), _wire_dt),
            pltpu.VMEM((N_LAYER * N_DEV, H, D), _wire_dt),
            pltpu.SemaphoreType.DMA((N_LAYER * 2, N_DEV - 1)),
            pltpu.SemaphoreType.DMA((N_LAYER * 2, N_DEV - 1)),
        ],
        compiler_params=pltpu.CompilerParams(collective_id=0),
    )(x, Win0, Wout0, Win1, Wout1, Win2, Wout2)
